# baseline (speedup 1.0000x reference)
import functools

import jax
import jax.numpy as jnp
from jax import lax
from jax.experimental import pallas as pl
from jax.experimental.pallas import tpu as pltpu
from jax.experimental.pallas import tpu_sc as plsc

_ROWS_OUT = 4095
_COLS = 8192
_OCOLS = _COLS + 1
_LANES = 16

_RPW = 128
_RB = 8
_SEG = 2048
_NSEG = _COLS // _SEG
_NGRP = _RPW // _RB
_NT = _NGRP * _NSEG
_NSLOT = 3
_JCH = _SEG // _LANES


def _make_sc_kernel():
    mesh = plsc.VectorSubcoreMesh(core_axis_name="c", subcore_axis_name="s")

    @functools.partial(
        pl.kernel,
        out_type=jax.ShapeDtypeStruct((_ROWS_OUT, _OCOLS), jnp.float32),
        mesh=mesh,
        compiler_params=pltpu.CompilerParams(needs_layout_passes=False),
        scratch_types=[
            pltpu.VMEM((_NSLOT, _RB, _SEG), jnp.float32),
            pltpu.VMEM((_NSLOT, _RB, _SEG), jnp.float32),
            pltpu.VMEM((_RPW, 128), jnp.float32),
            pltpu.SemaphoreType.DMA,
            pltpu.SemaphoreType.DMA,
            pltpu.SemaphoreType.DMA,
            pltpu.SemaphoreType.DMA,
            pltpu.SemaphoreType.DMA,
            pltpu.SemaphoreType.DMA,
        ],
    )
    def sc_excl_scan(x_hbm, out_hbm, ibuf, obuf, totbuf,
                     lds0, lds1, lds2, sts0, sts1, sts2):
        ld_sems = (lds0, lds1, lds2)
        st_sems = (sts0, sts1, sts2)
        cid = lax.axis_index("c")
        sid = lax.axis_index("s")
        wid = cid * 16 + sid
        wband = wid * _RPW

        def tick_gc(t):
            return t // _NSEG, t % _NSEG

        def start_load(t, slot):
            g, c = tick_gc(t)
            pltpu.async_copy(
                x_hbm.at[pl.ds(wband + g * _RB, _RB), pl.ds(c * _SEG, _SEG)],
                ibuf.at[slot],
                ld_sems[slot],
            )

        def wait_load(slot):
            pltpu.make_async_copy(
                x_hbm.at[pl.ds(0, _RB), pl.ds(0, _SEG)],
                ibuf.at[slot],
                ld_sems[slot],
            ).wait()

        def start_store(t, slot):
            g, c = tick_gc(t)
            pltpu.async_copy(
                obuf.at[slot],
                out_hbm.at[pl.ds(wband + g * _RB, _RB), pl.ds(c * _SEG, _SEG)],
                st_sems[slot],
            )

        def wait_store(slot):
            pltpu.make_async_copy(
                obuf.at[slot],
                out_hbm.at[pl.ds(0, _RB), pl.ds(0, _SEG)],
                st_sems[slot],
            ).wait()

        def compute(t, slot, cs):
            g, c = tick_gc(t)
            cs = tuple(
                jnp.where(c == 0, jnp.float32(0.0), cs[r]) for r in range(_RB))

            def jbody(j, carries):
                new = []
                for r in range(_RB):
                    v = ibuf[slot, r, pl.ds(j * _LANES, _LANES)]
                    s = plsc.cumsum(v)
                    obuf[slot, r, pl.ds(j * _LANES, _LANES)] = (s - v) + carries[r]
                    new.append(carries[r] + s[_LANES - 1])
                return tuple(new)

            cs = plsc.parallel_loop(0, _JCH, carry=cs, unroll=4)(jbody)

            @pl.when(c == _NSEG - 1)
            def _():
                for r in range(_RB):
                    totbuf[g * _RB + r, pl.ds(0, _LANES)] = jnp.broadcast_to(
                        cs[r], (_LANES,))

            return cs

        def turn(t, slot, cs, wait_st, issue_ld):
            wait_load(slot)
            if wait_st:
                wait_store(slot)
            cs = compute(t, slot, cs)
            start_store(t, slot)
            if issue_ld:
                @pl.when(t + _NSLOT < _NT)
                def _():
                    start_load(t + _NSLOT, slot)
            return cs

        for slot in range(_NSLOT):
            start_load(slot, slot)

        def cyc_body(q, cs):
            for s in range(_NSLOT):
                cs = turn(q * _NSLOT + s, s, cs, wait_st=True, issue_ld=True)
            return cs

        cs = (jnp.float32(0.0),) * _RB
        for s in range(_NSLOT):
            cs = turn(s, s, cs, wait_st=False, issue_ld=True)
        cs = lax.fori_loop(1, _NT // _NSLOT, cyc_body, cs)
        cs = turn(_NT - 1, (_NT - 1) % _NSLOT, cs, wait_st=True,
                  issue_ld=False)

        for t in range(_NT - _NSLOT, _NT):
            wait_store(t % _NSLOT)

        cstart = pl.multiple_of(_COLS + wid * 0, 128)
        pltpu.sync_copy(
            totbuf.at[:, :],
            out_hbm.at[pl.ds(wband, _RPW), pl.ds(cstart, 128)],
        )

    return sc_excl_scan


_sc_kernel = _make_sc_kernel()


@jax.jit
def kernel(x):
    return _sc_kernel(x)


# speedup vs baseline: 1.1169x; 1.1169x over previous
import functools

import jax
import jax.numpy as jnp
from jax import lax
from jax.experimental import pallas as pl
from jax.experimental.pallas import tpu as pltpu
from jax.experimental.pallas import tpu_sc as plsc

_ROWS_OUT = 4095
_COLS = 8192
_OCOLS = _COLS + 1
_LANES = 16

_RPW = 128
_RB = 8
_SEG = 2048
_NSEG = _COLS // _SEG
_NGRP = _RPW // _RB
_NT = _NGRP * _NSEG
_NSLOT = 3
_JCH = _SEG // _LANES


def _make_sc_kernel():
    mesh = plsc.VectorSubcoreMesh(core_axis_name="c", subcore_axis_name="s")

    @functools.partial(
        pl.kernel,
        out_type=jax.ShapeDtypeStruct((_ROWS_OUT, _OCOLS), jnp.float32),
        mesh=mesh,
        compiler_params=pltpu.CompilerParams(needs_layout_passes=False),
        scratch_types=[
            pltpu.VMEM((_NSLOT, _RB, _SEG), jnp.float32),
            pltpu.VMEM((_NSLOT, _RB, _SEG), jnp.float32),
            pltpu.VMEM((_RPW, 128), jnp.float32),
            pltpu.SemaphoreType.DMA,
            pltpu.SemaphoreType.DMA,
            pltpu.SemaphoreType.DMA,
            pltpu.SemaphoreType.DMA,
            pltpu.SemaphoreType.DMA,
            pltpu.SemaphoreType.DMA,
        ],
    )
    def sc_excl_scan(x_hbm, out_hbm, ibuf, obuf, totbuf,
                     lds0, lds1, lds2, sts0, sts1, sts2):
        ld_sems = (lds0, lds1, lds2)
        st_sems = (sts0, sts1, sts2)
        cid = lax.axis_index("c")
        sid = lax.axis_index("s")
        wid = cid * 16 + sid
        wband = wid * _RPW

        def tick_gc(t):
            return t // _NSEG, t % _NSEG

        def start_load(t, slot):
            g, c = tick_gc(t)
            pltpu.async_copy(
                x_hbm.at[pl.ds(wband + g * _RB, _RB), pl.ds(c * _SEG, _SEG)],
                ibuf.at[slot],
                ld_sems[slot],
            )

        def wait_load(slot):
            pltpu.make_async_copy(
                x_hbm.at[pl.ds(0, _RB), pl.ds(0, _SEG)],
                ibuf.at[slot],
                ld_sems[slot],
            ).wait()

        def start_store(t, slot):
            g, c = tick_gc(t)
            pltpu.async_copy(
                obuf.at[slot],
                out_hbm.at[pl.ds(wband + g * _RB, _RB), pl.ds(c * _SEG, _SEG)],
                st_sems[slot],
            )

        def wait_store(slot):
            pltpu.make_async_copy(
                obuf.at[slot],
                out_hbm.at[pl.ds(0, _RB), pl.ds(0, _SEG)],
                st_sems[slot],
            ).wait()

        def compute(t, slot, cs):
            g, c = tick_gc(t)
            cs = tuple(
                jnp.where(c == 0, jnp.float32(0.0), cs[r]) for r in range(_RB))

            def jbody(j, carries):
                new = []
                for r in range(_RB):
                    v = ibuf[slot, r, pl.ds(j * _LANES, _LANES)]
                    s = plsc.cumsum(v)
                    obuf[slot, r, pl.ds(j * _LANES, _LANES)] = (s - v) + carries[r]
                    new.append(carries[r] + s[_LANES - 1])
                return tuple(new)

            cs = plsc.parallel_loop(0, _JCH, carry=cs, unroll=2)(jbody)

            @pl.when(c == _NSEG - 1)
            def _():
                for r in range(_RB):
                    totbuf[g * _RB + r, pl.ds(0, _LANES)] = jnp.broadcast_to(
                        cs[r], (_LANES,))

            return cs

        def turn(t, slot, cs, wait_st, issue_ld):
            wait_load(slot)
            if wait_st:
                wait_store(slot)
            cs = compute(t, slot, cs)
            start_store(t, slot)
            if issue_ld:
                @pl.when(t + _NSLOT < _NT)
                def _():
                    start_load(t + _NSLOT, slot)
            return cs

        for slot in range(_NSLOT):
            start_load(slot, slot)

        def cyc_body(q, cs):
            for s in range(_NSLOT):
                cs = turn(q * _NSLOT + s, s, cs, wait_st=True, issue_ld=True)
            return cs

        cs = (jnp.float32(0.0),) * _RB
        for s in range(_NSLOT):
            cs = turn(s, s, cs, wait_st=False, issue_ld=True)
        cs = lax.fori_loop(1, _NT // _NSLOT, cyc_body, cs)
        cs = turn(_NT - 1, (_NT - 1) % _NSLOT, cs, wait_st=True,
                  issue_ld=False)

        for t in range(_NT - _NSLOT, _NT):
            wait_store(t % _NSLOT)

        cstart = pl.multiple_of(_COLS + wid * 0, 128)
        pltpu.sync_copy(
            totbuf.at[:, :],
            out_hbm.at[pl.ds(wband, _RPW), pl.ds(cstart, 128)],
        )

    return sc_excl_scan


_sc_kernel = _make_sc_kernel()


@jax.jit
def kernel(x):
    return _sc_kernel(x)


# speedup vs baseline: 1.1182x; 1.0011x over previous
import functools

import jax
import jax.numpy as jnp
from jax import lax
from jax.experimental import layout as jex_layout
from jax.experimental import pallas as pl
from jax.experimental.pallas import tpu as pltpu
from jax.experimental.pallas import tpu_sc as plsc

_ROWS_OUT = 4095
_COLS = 8192
_OCOLS = _COLS + 1
_LANES = 16

_RPW = 128
_RB = 8
_SEG = 2048
_NSEG = _COLS // _SEG
_NGRP = _RPW // _RB
_NT = _NGRP * _NSEG
_NSLOT = 3
_JCH = _SEG // _LANES


def _make_sc_kernel():
    mesh = plsc.VectorSubcoreMesh(core_axis_name="c", subcore_axis_name="s")

    @functools.partial(
        pl.kernel,
        out_type=jax.ShapeDtypeStruct((_ROWS_OUT, _OCOLS), jnp.float32),
        mesh=mesh,
        compiler_params=pltpu.CompilerParams(needs_layout_passes=False),
        scratch_types=[
            pltpu.VMEM((_NSLOT, _RB, _SEG), jnp.float32),
            pltpu.VMEM((_NSLOT, _RB, _SEG), jnp.float32),
            pltpu.VMEM((_RPW, 128), jnp.float32),
            pltpu.SemaphoreType.DMA,
            pltpu.SemaphoreType.DMA,
            pltpu.SemaphoreType.DMA,
            pltpu.SemaphoreType.DMA,
            pltpu.SemaphoreType.DMA,
            pltpu.SemaphoreType.DMA,
        ],
    )
    def sc_excl_scan(x_hbm, out_hbm, ibuf, obuf, totbuf,
                     lds0, lds1, lds2, sts0, sts1, sts2):
        ld_sems = (lds0, lds1, lds2)
        st_sems = (sts0, sts1, sts2)
        cid = lax.axis_index("c")
        sid = lax.axis_index("s")
        wid = cid * 16 + sid
        wband = wid * _RPW

        def tick_gc(t):
            return t // _NSEG, t % _NSEG

        def start_load(t, slot):
            g, c = tick_gc(t)
            pltpu.async_copy(
                x_hbm.at[pl.ds(wband + g * _RB, _RB), pl.ds(c * _SEG, _SEG)],
                ibuf.at[slot],
                ld_sems[slot],
            )

        def wait_load(slot):
            pltpu.make_async_copy(
                x_hbm.at[pl.ds(0, _RB), pl.ds(0, _SEG)],
                ibuf.at[slot],
                ld_sems[slot],
            ).wait()

        def start_store(t, slot):
            g, c = tick_gc(t)
            pltpu.async_copy(
                obuf.at[slot],
                out_hbm.at[pl.ds(wband + g * _RB, _RB), pl.ds(c * _SEG, _SEG)],
                st_sems[slot],
            )

        def wait_store(slot):
            pltpu.make_async_copy(
                obuf.at[slot],
                out_hbm.at[pl.ds(0, _RB), pl.ds(0, _SEG)],
                st_sems[slot],
            ).wait()

        def compute(t, slot, cs):
            g, c = tick_gc(t)
            cs = tuple(
                jnp.where(c == 0, jnp.float32(0.0), cs[r]) for r in range(_RB))

            def jbody(j, carries):
                new = []
                for r in range(_RB):
                    v = ibuf[slot, r, pl.ds(j * _LANES, _LANES)]
                    s = plsc.cumsum(v)
                    obuf[slot, r, pl.ds(j * _LANES, _LANES)] = (s - v) + carries[r]
                    new.append(carries[r] + s[_LANES - 1])
                return tuple(new)

            cs = plsc.parallel_loop(0, _JCH, carry=cs, unroll=2)(jbody)

            @pl.when(c == _NSEG - 1)
            def _():
                for r in range(_RB):
                    totbuf[g * _RB + r, pl.ds(0, _LANES)] = jnp.broadcast_to(
                        cs[r], (_LANES,))

            return cs

        def turn(t, slot, cs, wait_st, issue_ld):
            wait_load(slot)
            if wait_st:
                wait_store(slot)
            cs = compute(t, slot, cs)
            start_store(t, slot)
            if issue_ld:
                @pl.when(t + _NSLOT < _NT)
                def _():
                    start_load(t + _NSLOT, slot)
            return cs

        for slot in range(_NSLOT):
            start_load(slot, slot)

        def cyc_body(q, cs):
            for s in range(_NSLOT):
                cs = turn(q * _NSLOT + s, s, cs, wait_st=True, issue_ld=True)
            return cs

        cs = (jnp.float32(0.0),) * _RB
        for s in range(_NSLOT):
            cs = turn(s, s, cs, wait_st=False, issue_ld=True)
        cs = lax.fori_loop(1, _NT // _NSLOT, cyc_body, cs)
        cs = turn(_NT - 1, (_NT - 1) % _NSLOT, cs, wait_st=True,
                  issue_ld=False)

        for t in range(_NT - _NSLOT, _NT):
            wait_store(t % _NSLOT)

        cstart = pl.multiple_of(_COLS + wid * 0, 128)
        pltpu.sync_copy(
            totbuf.at[:, :],
            out_hbm.at[pl.ds(wband, _RPW), pl.ds(cstart, 128)],
        )

    return sc_excl_scan


_sc_kernel = _make_sc_kernel()


def _kernel_impl(x):
    return _sc_kernel(x)


@functools.lru_cache(maxsize=1)
def _jitted_kernel():
    fmt = jex_layout.Format(
        jex_layout.Layout(major_to_minor=(0, 1)),
        jax.sharding.SingleDeviceSharding(jax.devices()[0]),
    )
    return jax.jit(_kernel_impl, out_shardings=fmt)


def kernel(x):
    return _jitted_kernel()(x)


# speedup vs baseline: 2.2750x; 2.0345x over previous
import jax
import jax.numpy as jnp
from jax.experimental import pallas as pl

_ROWS_IN = 4096
_ROWS_OUT = 4095
_COLS = 8192
_CHUNK = 128
_NCHUNK = _COLS // _CHUNK
_BLK_R = 128


def _strict_upper(n, dtype):
    r = jax.lax.broadcasted_iota(jnp.int32, (n, n), 0)
    c = jax.lax.broadcasted_iota(jnp.int32, (n, n), 1)
    return (r < c).astype(dtype)


def _excl_cumsum_t_kernel(x_ref, o_ref):
    r = x_ref.shape[0]
    t128 = _strict_upper(_CHUNK, jnp.float32)
    t64 = _strict_upper(_NCHUNK, jnp.float32)

    x2 = x_ref[...].reshape(r * _NCHUNK, _CHUNK)
    excl_w = jnp.dot(x2, t128, preferred_element_type=jnp.float32)
    tots = jnp.sum(x2, axis=1).reshape(r, _NCHUNK)
    excl_t = jnp.dot(tots, t64, preferred_element_type=jnp.float32)

    out = excl_w.reshape(r, _NCHUNK, _CHUNK) + excl_t[:, :, None]
    o_ref[0:_COLS, :] = out.reshape(r, _COLS).T
    o_ref[_COLS:_COLS + 1, :] = (excl_t[:, _NCHUNK - 1]
                                 + tots[:, _NCHUNK - 1])[None, :]


@jax.jit
def kernel(x):
    grid = _ROWS_IN // _BLK_R
    t = pl.pallas_call(
        _excl_cumsum_t_kernel,
        grid=(grid,),
        in_specs=[pl.BlockSpec((_BLK_R, _COLS), lambda i: (i, 0))],
        out_specs=pl.BlockSpec((_COLS + 1, _BLK_R), lambda i: (0, i)),
        out_shape=jax.ShapeDtypeStruct((_COLS + 1, _ROWS_OUT), jnp.float32),
    )(x)
    return t.T


# speedup vs baseline: 2.2840x; 1.0040x over previous
import jax
import jax.numpy as jnp
from jax.experimental import pallas as pl

_ROWS_IN = 4096
_ROWS_OUT = 4095
_COLS = 8192
_CHUNK = 128
_NCHUNK = _COLS // _CHUNK
_BLK_R = 256


def _strict_upper(n, dtype):
    r = jax.lax.broadcasted_iota(jnp.int32, (n, n), 0)
    c = jax.lax.broadcasted_iota(jnp.int32, (n, n), 1)
    return (r < c).astype(dtype)


def _excl_cumsum_t_kernel(x_ref, o_ref):
    r = x_ref.shape[0]
    t128 = _strict_upper(_CHUNK, jnp.float32)
    t64 = _strict_upper(_NCHUNK, jnp.float32)

    x2 = x_ref[...].reshape(r * _NCHUNK, _CHUNK)
    excl_w = jnp.dot(x2, t128, preferred_element_type=jnp.float32)
    tots = jnp.sum(x2, axis=1).reshape(r, _NCHUNK)
    excl_t = jnp.dot(tots, t64, preferred_element_type=jnp.float32)

    out = excl_w.reshape(r, _NCHUNK, _CHUNK) + excl_t[:, :, None]
    o_ref[0:_COLS, :] = out.reshape(r, _COLS).T
    o_ref[_COLS:_COLS + 1, :] = (excl_t[:, _NCHUNK - 1]
                                 + tots[:, _NCHUNK - 1])[None, :]


@jax.jit
def kernel(x):
    grid = _ROWS_IN // _BLK_R
    t = pl.pallas_call(
        _excl_cumsum_t_kernel,
        grid=(grid,),
        in_specs=[pl.BlockSpec((_BLK_R, _COLS), lambda i: (i, 0))],
        out_specs=pl.BlockSpec((_COLS + 1, _BLK_R), lambda i: (0, i)),
        out_shape=jax.ShapeDtypeStruct((_COLS + 1, _ROWS_OUT), jnp.float32),
    )(x)
    return t.T
